# baseline (speedup 1.0000x reference)
import functools

import jax
import jax.numpy as jnp
import numpy as np
from jax import lax
from jax.experimental import pallas as pl
from jax.experimental.pallas import tpu as pltpu
from jax.experimental.pallas import tpu_sc as plsc

N = 4096
K = 64
B = 64
H = 128
G = 50
T = 6
CUTOFF = 10.0
GRID = 2048
INV = (GRID - 1) / CUTOFF

NC = 2
NS = 16
NW = NC * NS
NODES_PER_W = N // NW
GROUP = 2
EG = GROUP * K
NGRP = NODES_PER_W // GROUP


def _ssp(x):
    return jax.nn.softplus(x) - jnp.log(2.0)


def _tab_kernel(w1_ref, b1_ref, w2_ref, b2_ref, out_ref):
    dg = (jax.lax.broadcasted_iota(jnp.int32, (GRID, G), 0).astype(jnp.float32)
          * (CUTOFF / (GRID - 1)))
    off = (jax.lax.broadcasted_iota(jnp.int32, (GRID, G), 1).astype(jnp.float32)
           * (CUTOFF / (G - 1)))
    coeff = -0.5 / (CUTOFF / (G - 1)) ** 2
    ea = jnp.exp(coeff * (dg - off) ** 2)
    w1 = w1_ref[...].reshape(G, H)
    w2 = w2_ref[...].reshape(H, H)
    f = _ssp(jnp.dot(ea, w1, preferred_element_type=jnp.float32)
             + b1_ref[...].reshape(1, H))
    m = (jnp.dot(f, w2, preferred_element_type=jnp.float32)
         + b2_ref[...].reshape(1, H))
    dcol = (jax.lax.broadcasted_iota(jnp.int32, (GRID, 1), 0).astype(jnp.float32)
            * (CUTOFF / (GRID - 1)))
    c = 0.5 * (jnp.cos(dcol * (np.pi / CUTOFF)) + 1.0)
    v = m * c
    vshift = jnp.concatenate([v[1:, :], v[GRID - 1:, :]], axis=0)
    s = vshift - v
    out_ref[...] = jnp.concatenate([v, s], axis=1).reshape(1, GRID, 2 * H)


def _build_tables(mlp_w1, mlp_b1, mlp_w2, mlp_b2):
    return pl.pallas_call(
        _tab_kernel,
        grid=(T,),
        in_specs=[
            pl.BlockSpec((1, G, H), lambda t: (t, 0, 0)),
            pl.BlockSpec((1, 1, H), lambda t: (t, 0, 0)),
            pl.BlockSpec((1, H, H), lambda t: (t, 0, 0)),
            pl.BlockSpec((1, 1, H), lambda t: (t, 0, 0)),
        ],
        out_specs=pl.BlockSpec((1, GRID, 2 * H), lambda t: (t, 0, 0)),
        out_shape=jax.ShapeDtypeStruct((T, GRID, 2 * H), jnp.float32),
    )(mlp_w1, mlp_b1.reshape(T, 1, H), mlp_w2, mlp_b2.reshape(T, 1, H))


def _pre_kernel(z_ref, emb_ref, l1_ref, h_ref, xq_ref):
    z = z_ref[...]
    oh = (z == jax.lax.broadcasted_iota(jnp.int32, (N, 100), 1)).astype(jnp.float32)
    h = jnp.dot(oh, emb_ref[...], preferred_element_type=jnp.float32)
    h_ref[...] = h
    xq_ref[...] = jnp.dot(h, l1_ref[...].reshape(H, H),
                          preferred_element_type=jnp.float32)


def _pre(z2d, emb, lin1_w):
    return pl.pallas_call(
        _pre_kernel,
        in_specs=[
            pl.BlockSpec((N, 1), lambda: (0, 0)),
            pl.BlockSpec((100, H), lambda: (0, 0)),
            pl.BlockSpec((1, H, H), lambda: (0, 0, 0)),
        ],
        out_specs=[
            pl.BlockSpec((N, H), lambda: (0, 0)),
            pl.BlockSpec((N, H), lambda: (0, 0)),
        ],
        out_shape=[
            jax.ShapeDtypeStruct((N, H), jnp.float32),
            jax.ShapeDtypeStruct((N, H), jnp.float32),
        ],
    )(z2d, emb, lin1_w[0:1])


def _node_kernel(agg_ref, h_ref, l2_ref, b2_ref, il_ref, ib_ref, l1n_ref,
                 hn_ref, xqn_ref):
    agg = agg_ref[...]
    xo = _ssp(jnp.dot(agg, l2_ref[...].reshape(H, H),
                      preferred_element_type=jnp.float32)
              + b2_ref[...].reshape(1, H))
    xo = (jnp.dot(xo, il_ref[...].reshape(H, H),
                  preferred_element_type=jnp.float32)
          + ib_ref[...].reshape(1, H))
    hn = h_ref[...] + xo
    hn_ref[...] = hn
    xqn_ref[...] = jnp.dot(hn, l1n_ref[...].reshape(H, H),
                           preferred_element_type=jnp.float32)


def _node(agg, h, l2, b2, il, ib, l1n):
    return pl.pallas_call(
        _node_kernel,
        in_specs=[
            pl.BlockSpec((N, H), lambda: (0, 0)),
            pl.BlockSpec((N, H), lambda: (0, 0)),
            pl.BlockSpec((1, H, H), lambda: (0, 0, 0)),
            pl.BlockSpec((1, 1, H), lambda: (0, 0, 0)),
            pl.BlockSpec((1, H, H), lambda: (0, 0, 0)),
            pl.BlockSpec((1, 1, H), lambda: (0, 0, 0)),
            pl.BlockSpec((1, H, H), lambda: (0, 0, 0)),
        ],
        out_specs=[
            pl.BlockSpec((N, H), lambda: (0, 0)),
            pl.BlockSpec((N, H), lambda: (0, 0)),
        ],
        out_shape=[
            jax.ShapeDtypeStruct((N, H), jnp.float32),
            jax.ShapeDtypeStruct((N, H), jnp.float32),
        ],
    )(agg, h, l2, b2, il, ib, l1n)


def _pool_kernel(batch_ref, h_ref, pw_ref, pb_ref, out_ref):
    bm = (jax.lax.broadcasted_iota(jnp.int32, (B, N), 0)
          == batch_ref[...]).astype(jnp.float32)
    sums = jnp.dot(bm, h_ref[...], preferred_element_type=jnp.float32)
    counts = jnp.sum(bm, axis=1, keepdims=True)
    pooled = jnp.where(counts > 0, sums / jnp.maximum(counts, 1.0), 0.0)
    out_ref[...] = jnp.dot(pooled, pw_ref[...],
                           preferred_element_type=jnp.float32) + pb_ref[...]


def _pool(batch_row, h, proj_w, proj_b2d):
    return pl.pallas_call(
        _pool_kernel,
        in_specs=[
            pl.BlockSpec((1, N), lambda: (0, 0)),
            pl.BlockSpec((N, H), lambda: (0, 0)),
            pl.BlockSpec((H, H), lambda: (0, 0)),
            pl.BlockSpec((1, H), lambda: (0, 0)),
        ],
        out_specs=pl.BlockSpec((B, H), lambda: (0, 0)),
        out_shape=jax.ShapeDtypeStruct((B, H), jnp.float32),
    )(batch_row, h, proj_w, proj_b2d)


_SPLAT_DNUMS = lax.GatherDimensionNumbers(
    offset_dims=(), collapsed_slice_dims=(0,), start_index_map=(0,))


def _agg_body(xq_hbm, nbr_hbm, ew_hbm, tab_hbm, out_hbm,
              nbr_v, g_v, al_v, xrows, trows, aggv, sem_e, sem_x):
    wid = lax.axis_index("s") * NC + lax.axis_index("c")
    base_e = wid * NODES_PER_W * K

    def issue_edges(grp):
        p = grp % 2
        e0 = base_e + (grp % NGRP) * EG
        pltpu.async_copy(nbr_hbm.at[pl.ds(e0, EG)],
                         nbr_v.at[pl.ds(p * EG, EG)], sem_e)
        pltpu.async_copy(ew_hbm.at[pl.ds(e0, EG)],
                         al_v.at[pl.ds(p * EG, EG)], sem_e)

    def drain_edges(grp):
        p = grp % 2
        pltpu.make_async_copy(nbr_hbm.at[pl.ds(0, EG)],
                              nbr_v.at[pl.ds(p * EG, EG)], sem_e).wait()
        pltpu.make_async_copy(ew_hbm.at[pl.ds(0, EG)],
                              al_v.at[pl.ds(p * EG, EG)], sem_e).wait()

    def quant_and_issue(grp):
        p = grp % 2
        boff = p * EG
        for ci in range(EG // 16):
            t = al_v[pl.ds(boff + ci * 16, 16)] * INV
            g = t.astype(jnp.int32)
            al_v[pl.ds(boff + ci * 16, 16)] = t - g.astype(jnp.float32)
            g_v[pl.ds(boff + ci * 16, 16)] = g
        pltpu.async_copy(xq_hbm.at[nbr_v.at[pl.ds(boff, EG)]],
                         xrows.at[pl.ds(boff, EG), :], sem_x)
        pltpu.async_copy(tab_hbm.at[g_v.at[pl.ds(boff, EG)]],
                         trows.at[pl.ds(boff, EG), :], sem_x)

    def compute(grp):
        p = grp % 2
        boff = p * EG
        pltpu.make_async_copy(xq_hbm.at[nbr_v.at[pl.ds(0, EG)]],
                              xrows.at[pl.ds(0, EG), :], sem_x).wait()
        pltpu.make_async_copy(tab_hbm.at[g_v.at[pl.ds(0, EG)]],
                              trows.at[pl.ds(0, EG), :], sem_x).wait()
        for ni in range(GROUP):
            def edge_body(k, acc):
                e = boff + ni * K + k
                kbase = e & ~15
                chunk = al_v[pl.ds(kbase, 16)]
                als = lax.gather(
                    chunk,
                    (jnp.full((16,), 0, jnp.int32) + (e - kbase))[:, None],
                    _SPLAT_DNUMS, slice_sizes=(1,),
                    mode=lax.GatherScatterMode.PROMISE_IN_BOUNDS)
                new = []
                for s8 in range(8):
                    v = trows[e, pl.ds(s8 * 16, 16)]
                    sl = trows[e, pl.ds(H + s8 * 16, 16)]
                    x = xrows[e, pl.ds(s8 * 16, 16)]
                    new.append(acc[s8] + (v + als * sl) * x)
                return tuple(new)

            acc0 = tuple(jnp.zeros((16,), jnp.float32) for _ in range(8))
            acc = lax.fori_loop(0, K, edge_body, acc0, unroll=8)
            for s8 in range(8):
                aggv[ni, pl.ds(s8 * 16, 16)] = acc[s8]
        node0 = wid * NODES_PER_W + grp * GROUP
        pltpu.sync_copy(aggv, out_hbm.at[pl.ds(node0, GROUP), :])

    issue_edges(0)
    drain_edges(0)
    quant_and_issue(0)
    issue_edges(1)

    def grp_body(g, _):
        drain_edges(g + 1)
        quant_and_issue(g + 1)
        compute(g)
        issue_edges(g + 2)
        return 0

    lax.fori_loop(0, NGRP - 1, grp_body, 0)
    compute(NGRP - 1)
    drain_edges(NGRP)


@functools.partial(jax.jit, static_argnums=())
def _agg(xq, nbr_flat, ew_flat, tab):
    mesh = plsc.VectorSubcoreMesh(core_axis_name="c", subcore_axis_name="s")
    body = functools.partial(
        pl.kernel,
        out_type=jax.ShapeDtypeStruct((N, H), jnp.float32),
        mesh=mesh,
        scratch_types=[
            pltpu.VMEM((2 * EG,), jnp.int32),
            pltpu.VMEM((2 * EG,), jnp.int32),
            pltpu.VMEM((2 * EG,), jnp.float32),
            pltpu.VMEM((2 * EG, H), jnp.float32),
            pltpu.VMEM((2 * EG, 2 * H), jnp.float32),
            pltpu.VMEM((GROUP, H), jnp.float32),
            pltpu.SemaphoreType.DMA,
            pltpu.SemaphoreType.DMA,
        ],
    )(_agg_body)
    return body(xq, nbr_flat, ew_flat, tab)


def kernel(z, pos, batch, emb, mlp_w1, mlp_b1, mlp_w2, mlp_b2,
           lin1_w, lin2_w, lin2_b, ilin_w, ilin_b, proj_w, proj_b):
    W = 512
    batch_i = batch.astype(jnp.int32)

    def _radius_windowed(_):
        starts = jnp.searchsorted(batch_i, jnp.arange(B, dtype=jnp.int32))
        lo = jnp.minimum(starts[batch_i], N - W).astype(jnp.int32)
        ci = lo[:, None] + jnp.arange(W, dtype=jnp.int32)[None, :]
        pb = pos[ci]
        bb = batch_i[ci]
        d2w = jnp.sum((pos[:, None, :] - pb) ** 2, axis=-1)
        distw = jnp.sqrt(jnp.maximum(d2w, 1e-12))
        rows = jnp.arange(N, dtype=jnp.int32)[:, None]
        validw = (bb == batch_i[:, None]) & (ci != rows) & (distw < CUTOFF)
        scoresw = jnp.where(validw, -distw, -1e9)
        valsw, idxw = jax.lax.top_k(scoresw, K)
        nbrw = jnp.take_along_axis(ci, idxw, axis=1)
        maskw = valsw > -1e8
        eww = jnp.where(maskw, -valsw, CUTOFF)
        return nbrw.reshape(-1), eww.reshape(-1)

    def _radius_full(_):
        d2 = jnp.sum((pos[:, None, :] - pos[None, :, :]) ** 2, axis=-1)
        dist = jnp.sqrt(jnp.maximum(d2, 1e-12))
        same = batch[:, None] == batch[None, :]
        valid = same & (~jnp.eye(N, dtype=bool)) & (dist < CUTOFF)
        scores = jnp.where(valid, -dist, -1e9)
        vals, nbr = jax.lax.top_k(scores, K)
        mask = vals > -1e8
        return (nbr.reshape(-1).astype(jnp.int32),
                jnp.where(mask, -vals, CUTOFF).reshape(-1))

    sizes = jnp.sum(batch_i[:, None] == jnp.arange(B, dtype=jnp.int32)[None, :],
                    axis=0)
    nbr_flat, ew_flat = lax.cond(jnp.max(sizes) <= W,
                                 _radius_windowed, _radius_full, 0)

    tab = _build_tables(mlp_w1, mlp_b1, mlp_w2, mlp_b2)
    h, xq = _pre(z.reshape(N, 1).astype(jnp.int32), emb, lin1_w)
    for t in range(T):
        agg = _agg(xq, nbr_flat, ew_flat, tab[t])
        l1n = lin1_w[t + 1:t + 2] if t + 1 < T else lin1_w[0:1]
        h, xq = _node(agg, h, lin2_w[t:t + 1], lin2_b[t:t + 1].reshape(1, 1, H),
                      ilin_w[t:t + 1], ilin_b[t:t + 1].reshape(1, 1, H), l1n)
    return _pool(batch.reshape(1, N).astype(jnp.int32), h, proj_w,
                 proj_b.reshape(1, H))


# speedup vs baseline: 4.0329x; 4.0329x over previous
import functools

import jax
import jax.numpy as jnp
import numpy as np
from jax import lax
from jax.experimental import pallas as pl
from jax.experimental.pallas import tpu as pltpu
from jax.experimental.pallas import tpu_sc as plsc

N = 4096
K = 64
B = 64
H = 128
G = 50
T = 6
CUTOFF = 10.0
GRID = 8192
INV = (GRID - 1) / CUTOFF

NC = 2
NS = 16
NW = NC * NS
NODES_PER_W = N // NW
GROUP = 2
EG = GROUP * K
NGRP = NODES_PER_W // GROUP


def _ssp(x):
    return jax.nn.softplus(x) - jnp.log(2.0)


def _tab_kernel(w1_ref, b1_ref, w2_ref, b2_ref, out_ref):
    dg = (jax.lax.broadcasted_iota(jnp.int32, (GRID, G), 0).astype(jnp.float32)
          * (CUTOFF / (GRID - 1)))
    off = (jax.lax.broadcasted_iota(jnp.int32, (GRID, G), 1).astype(jnp.float32)
           * (CUTOFF / (G - 1)))
    coeff = -0.5 / (CUTOFF / (G - 1)) ** 2
    ea = jnp.exp(coeff * (dg - off) ** 2)
    w1 = w1_ref[...].reshape(G, H)
    w2 = w2_ref[...].reshape(H, H)
    f = _ssp(jnp.dot(ea, w1, preferred_element_type=jnp.float32)
             + b1_ref[...].reshape(1, H))
    m = (jnp.dot(f, w2, preferred_element_type=jnp.float32)
         + b2_ref[...].reshape(1, H))
    dcol = (jax.lax.broadcasted_iota(jnp.int32, (GRID, 1), 0).astype(jnp.float32)
            * (CUTOFF / (GRID - 1)))
    c = 0.5 * (jnp.cos(dcol * (np.pi / CUTOFF)) + 1.0)
    out_ref[...] = (m * c).reshape(1, GRID, H)


def _build_tables(mlp_w1, mlp_b1, mlp_w2, mlp_b2):
    return pl.pallas_call(
        _tab_kernel,
        grid=(T,),
        in_specs=[
            pl.BlockSpec((1, G, H), lambda t: (t, 0, 0)),
            pl.BlockSpec((1, 1, H), lambda t: (t, 0, 0)),
            pl.BlockSpec((1, H, H), lambda t: (t, 0, 0)),
            pl.BlockSpec((1, 1, H), lambda t: (t, 0, 0)),
        ],
        out_specs=pl.BlockSpec((1, GRID, H), lambda t: (t, 0, 0)),
        out_shape=jax.ShapeDtypeStruct((T, GRID, H), jnp.float32),
    )(mlp_w1, mlp_b1.reshape(T, 1, H), mlp_w2, mlp_b2.reshape(T, 1, H))


def _pre_kernel(z_ref, emb_ref, l1_ref, h_ref, xq_ref):
    z = z_ref[...]
    oh = (z == jax.lax.broadcasted_iota(jnp.int32, (N, 100), 1)).astype(jnp.float32)
    h = jnp.dot(oh, emb_ref[...], preferred_element_type=jnp.float32)
    h_ref[...] = h
    xq_ref[...] = jnp.dot(h, l1_ref[...].reshape(H, H),
                          preferred_element_type=jnp.float32)


def _pre(z2d, emb, lin1_w):
    return pl.pallas_call(
        _pre_kernel,
        in_specs=[
            pl.BlockSpec((N, 1), lambda: (0, 0)),
            pl.BlockSpec((100, H), lambda: (0, 0)),
            pl.BlockSpec((1, H, H), lambda: (0, 0, 0)),
        ],
        out_specs=[
            pl.BlockSpec((N, H), lambda: (0, 0)),
            pl.BlockSpec((N, H), lambda: (0, 0)),
        ],
        out_shape=[
            jax.ShapeDtypeStruct((N, H), jnp.float32),
            jax.ShapeDtypeStruct((N, H), jnp.float32),
        ],
    )(z2d, emb, lin1_w[0:1])


def _node_kernel(agg_ref, h_ref, l2_ref, b2_ref, il_ref, ib_ref, l1n_ref,
                 hn_ref, xqn_ref):
    agg = agg_ref[...]
    xo = _ssp(jnp.dot(agg, l2_ref[...].reshape(H, H),
                      preferred_element_type=jnp.float32)
              + b2_ref[...].reshape(1, H))
    xo = (jnp.dot(xo, il_ref[...].reshape(H, H),
                  preferred_element_type=jnp.float32)
          + ib_ref[...].reshape(1, H))
    hn = h_ref[...] + xo
    hn_ref[...] = hn
    xqn_ref[...] = jnp.dot(hn, l1n_ref[...].reshape(H, H),
                           preferred_element_type=jnp.float32)


def _node(agg, h, l2, b2, il, ib, l1n):
    return pl.pallas_call(
        _node_kernel,
        in_specs=[
            pl.BlockSpec((N, H), lambda: (0, 0)),
            pl.BlockSpec((N, H), lambda: (0, 0)),
            pl.BlockSpec((1, H, H), lambda: (0, 0, 0)),
            pl.BlockSpec((1, 1, H), lambda: (0, 0, 0)),
            pl.BlockSpec((1, H, H), lambda: (0, 0, 0)),
            pl.BlockSpec((1, 1, H), lambda: (0, 0, 0)),
            pl.BlockSpec((1, H, H), lambda: (0, 0, 0)),
        ],
        out_specs=[
            pl.BlockSpec((N, H), lambda: (0, 0)),
            pl.BlockSpec((N, H), lambda: (0, 0)),
        ],
        out_shape=[
            jax.ShapeDtypeStruct((N, H), jnp.float32),
            jax.ShapeDtypeStruct((N, H), jnp.float32),
        ],
    )(agg, h, l2, b2, il, ib, l1n)


def _pool_kernel(batch_ref, h_ref, pw_ref, pb_ref, out_ref):
    bm = (jax.lax.broadcasted_iota(jnp.int32, (B, N), 0)
          == batch_ref[...]).astype(jnp.float32)
    sums = jnp.dot(bm, h_ref[...], preferred_element_type=jnp.float32)
    counts = jnp.sum(bm, axis=1, keepdims=True)
    pooled = jnp.where(counts > 0, sums / jnp.maximum(counts, 1.0), 0.0)
    out_ref[...] = jnp.dot(pooled, pw_ref[...],
                           preferred_element_type=jnp.float32) + pb_ref[...]


def _pool(batch_row, h, proj_w, proj_b2d):
    return pl.pallas_call(
        _pool_kernel,
        in_specs=[
            pl.BlockSpec((1, N), lambda: (0, 0)),
            pl.BlockSpec((N, H), lambda: (0, 0)),
            pl.BlockSpec((H, H), lambda: (0, 0)),
            pl.BlockSpec((1, H), lambda: (0, 0)),
        ],
        out_specs=pl.BlockSpec((B, H), lambda: (0, 0)),
        out_shape=jax.ShapeDtypeStruct((B, H), jnp.float32),
    )(batch_row, h, proj_w, proj_b2d)


_SPLAT_DNUMS = lax.GatherDimensionNumbers(
    offset_dims=(), collapsed_slice_dims=(0,), start_index_map=(0,))


def _agg_body(xq_hbm, nbr_hbm, ew_hbm, tab_hbm, out_hbm,
              nbr_v, g_v, al_v, xrows, trows, aggv, sem_e, sem_x):
    wid = lax.axis_index("s") * NC + lax.axis_index("c")
    base_e = wid * NODES_PER_W * K

    def issue_edges(grp):
        p = grp % 2
        e0 = base_e + (grp % NGRP) * EG
        pltpu.async_copy(nbr_hbm.at[pl.ds(e0, EG)],
                         nbr_v.at[pl.ds(p * EG, EG)], sem_e)
        pltpu.async_copy(ew_hbm.at[pl.ds(e0, EG)],
                         al_v.at[pl.ds(p * EG, EG)], sem_e)

    def drain_edges(grp):
        p = grp % 2
        pltpu.make_async_copy(nbr_hbm.at[pl.ds(0, EG)],
                              nbr_v.at[pl.ds(p * EG, EG)], sem_e).wait()
        pltpu.make_async_copy(ew_hbm.at[pl.ds(0, EG)],
                              al_v.at[pl.ds(p * EG, EG)], sem_e).wait()

    def quant_and_issue(grp):
        p = grp % 2
        boff = p * EG
        for ci in range(EG // 16):
            t = al_v[pl.ds(boff + ci * 16, 16)] * INV + 0.5
            g_v[pl.ds(boff + ci * 16, 16)] = t.astype(jnp.int32)
        pltpu.async_copy(xq_hbm.at[nbr_v.at[pl.ds(boff, EG)]],
                         xrows.at[pl.ds(boff, EG), :], sem_x)
        pltpu.async_copy(tab_hbm.at[g_v.at[pl.ds(boff, EG)]],
                         trows.at[pl.ds(boff, EG), :], sem_x)

    def compute(grp):
        p = grp % 2
        boff = p * EG
        pltpu.make_async_copy(xq_hbm.at[nbr_v.at[pl.ds(0, EG)]],
                              xrows.at[pl.ds(0, EG), :], sem_x).wait()
        pltpu.make_async_copy(tab_hbm.at[g_v.at[pl.ds(0, EG)]],
                              trows.at[pl.ds(0, EG), :], sem_x).wait()
        for ni in range(GROUP):
            def edge_body(k, acc):
                e = boff + ni * K + k
                new = []
                for s8 in range(8):
                    v = trows[e, pl.ds(s8 * 16, 16)]
                    x = xrows[e, pl.ds(s8 * 16, 16)]
                    new.append(acc[s8] + v * x)
                return tuple(new)

            acc0 = tuple(jnp.zeros((16,), jnp.float32) for _ in range(8))
            acc = lax.fori_loop(0, K, edge_body, acc0, unroll=8)
            for s8 in range(8):
                aggv[ni, pl.ds(s8 * 16, 16)] = acc[s8]
        node0 = wid * NODES_PER_W + grp * GROUP
        pltpu.sync_copy(aggv, out_hbm.at[pl.ds(node0, GROUP), :])

    issue_edges(0)
    drain_edges(0)
    quant_and_issue(0)
    issue_edges(1)

    def grp_body(g, _):
        drain_edges(g + 1)
        quant_and_issue(g + 1)
        compute(g)
        issue_edges(g + 2)
        return 0

    lax.fori_loop(0, NGRP - 1, grp_body, 0)
    compute(NGRP - 1)
    drain_edges(NGRP)


@functools.partial(jax.jit, static_argnums=())
def _agg(xq, nbr_flat, ew_flat, tab):
    mesh = plsc.VectorSubcoreMesh(core_axis_name="c", subcore_axis_name="s")
    body = functools.partial(
        pl.kernel,
        out_type=jax.ShapeDtypeStruct((N, H), jnp.float32),
        mesh=mesh,
        scratch_types=[
            pltpu.VMEM((2 * EG,), jnp.int32),
            pltpu.VMEM((2 * EG,), jnp.int32),
            pltpu.VMEM((2 * EG,), jnp.float32),
            pltpu.VMEM((2 * EG, H), jnp.float32),
            pltpu.VMEM((2 * EG, H), jnp.float32),
            pltpu.VMEM((GROUP, H), jnp.float32),
            pltpu.SemaphoreType.DMA,
            pltpu.SemaphoreType.DMA,
        ],
    )(_agg_body)
    return body(xq, nbr_flat, ew_flat, tab)


def kernel(z, pos, batch, emb, mlp_w1, mlp_b1, mlp_w2, mlp_b2,
           lin1_w, lin2_w, lin2_b, ilin_w, ilin_b, proj_w, proj_b):
    d2 = jnp.sum((pos[:, None, :] - pos[None, :, :]) ** 2, axis=-1)
    dist = jnp.sqrt(jnp.maximum(d2, 1e-12))
    same = batch[:, None] == batch[None, :]
    valid = same & (~jnp.eye(N, dtype=bool)) & (dist < CUTOFF)
    scores = jnp.where(valid, -dist, -1e9)
    vals, nbr = jax.lax.top_k(scores, K)
    mask = vals > -1e8
    ew_flat = jnp.where(mask, -vals, CUTOFF).reshape(-1)
    nbr_flat = nbr.reshape(-1).astype(jnp.int32)

    tab = _build_tables(mlp_w1, mlp_b1, mlp_w2, mlp_b2)
    h, xq = _pre(z.reshape(N, 1).astype(jnp.int32), emb, lin1_w)
    for t in range(T):
        agg = _agg(xq, nbr_flat, ew_flat, tab[t])
        l1n = lin1_w[t + 1:t + 2] if t + 1 < T else lin1_w[0:1]
        h, xq = _node(agg, h, lin2_w[t:t + 1], lin2_b[t:t + 1].reshape(1, 1, H),
                      ilin_w[t:t + 1], ilin_b[t:t + 1].reshape(1, 1, H), l1n)
    return _pool(batch.reshape(1, N).astype(jnp.int32), h, proj_w,
                 proj_b.reshape(1, H))
